# baseline (speedup 1.0000x reference)
import jax
import jax.numpy as jnp
from jax.experimental import pallas as pl
from jax.experimental.pallas import tpu as pltpu

B = 16384
H = 64
OUT = 64
NB = 4
LN_EPS = 1e-5
BN_EPS = 1e-5
P = 2 * H
BP = B // 2
TILE = 4096


def _body(x_ref, xb_ref, lnw_ref, lnb_ref, w1_ref, b1_ref, bnw_ref, bnb_ref,
          w2_ref, b2_ref, fcw_ref, fcb_ref, o_ref, w1m_ref, wp_ref, vec_ref):
    lane = jax.lax.broadcasted_iota(jnp.int32, (P, P), 1)
    sub = jax.lax.broadcasted_iota(jnp.int32, (P, P), 0)

    @pl.when(pl.program_id(0) == 0)
    def _setup():
        bmask = (((lane ^ sub) & H) == 0).astype(jnp.float32)
        dmask = (lane == sub).astype(jnp.float32)
        m = bmask * (1.0 / H)

        def bdiag(w):
            t = jnp.concatenate([w, w], axis=1)
            return jnp.concatenate([t, t], axis=0) * bmask

        def diag(v):
            return jnp.broadcast_to(v, (P, P)) * dmask

        def tile2(v):
            return jnp.concatenate([v, v], axis=1)

        wp_ref[NB] = bdiag(fcw_ref[...])
        vec_ref[2 * NB:2 * NB + 1, :] = tile2(fcb_ref[...])

        s = 1.0 / jnp.sqrt(1.0 + BN_EPS)
        for i in range(NB):
            gp = tile2(bnw_ref[i:i + 1, :]) * s
            w1p0g = jnp.dot(bdiag(w1_ref[i]), diag(gp),
                            preferred_element_type=jnp.float32)
            w1g = jnp.dot(diag(tile2(lnw_ref[i:i + 1, :])), w1p0g,
                          preferred_element_type=jnp.float32)
            c = jnp.sum(w1g, axis=0, keepdims=True)
            w1m_ref[i] = jnp.concatenate([w1g - m * c, m], axis=1)
            wp_ref[i] = bdiag(w2_ref[i])
            d = jnp.dot(tile2(lnb_ref[i:i + 1, :]), w1p0g,
                        preferred_element_type=jnp.float32)
            d = d + tile2(b1_ref[i:i + 1, :]) * gp + tile2(bnb_ref[i:i + 1, :])
            vec_ref[i:i + 1, :] = d
            vec_ref[NB + i:NB + i + 1, :] = tile2(b2_ref[i:i + 1, :])

    x = jnp.concatenate([x_ref[0], xb_ref[0]], axis=1)
    m = w1m_ref[0][:, P:2 * P]
    for i in range(NB):
        tm = jnp.dot(x, w1m_ref[i], preferred_element_type=jnp.float32)
        q = jnp.dot(x * x, m, preferred_element_type=jnp.float32)
        t = tm[:, 0:P]
        mu = tm[:, P:2 * P]
        r = jax.lax.rsqrt(q - mu * mu + LN_EPS)
        h = r * t + vec_ref[i:i + 1, :]
        h = jnp.maximum(h, 0.0)
        h = jnp.dot(h, wp_ref[i], preferred_element_type=jnp.float32)
        x = x + h + vec_ref[NB + i:NB + i + 1, :]
    o = jnp.dot(x, wp_ref[NB], preferred_element_type=jnp.float32)
    o = o + vec_ref[2 * NB:2 * NB + 1, :]
    o_ref[0] = o[:, 0:H]
    o_ref[1] = o[:, H:P]


def kernel(x, ln_w, ln_b, w1, b1, bn_w, bn_b, w2, b2, fc_w, fc_b):
    full = lambda *shape: pl.BlockSpec(shape, lambda i: (0,) * len(shape))
    x3 = x.reshape(2, BP, H)
    out = pl.pallas_call(
        _body,
        grid=(BP // TILE,),
        in_specs=[
            pl.BlockSpec((1, TILE, H), lambda i: (0, i, 0)),
            pl.BlockSpec((1, TILE, H), lambda i: (1, i, 0)),
            full(NB, H), full(NB, H), full(NB, H, H), full(NB, H),
            full(NB, H), full(NB, H), full(NB, H, H), full(NB, H),
            full(H, OUT), full(1, OUT),
        ],
        out_specs=pl.BlockSpec((2, TILE, H), lambda i: (0, i, 0)),
        out_shape=jax.ShapeDtypeStruct((2, BP, OUT), jnp.float32),
        scratch_shapes=[
            pltpu.VMEM((NB, P, 2 * P), jnp.float32),
            pltpu.VMEM((NB + 1, P, P), jnp.float32),
            pltpu.VMEM((2 * NB + 1, P), jnp.float32),
        ],
    )(x3, x3, ln_w, ln_b, w1, b1, bn_w, bn_b, w2, b2, fc_w,
      fc_b.reshape(1, OUT))
    return out.reshape(B, OUT)


# speedup vs baseline: 1.0086x; 1.0086x over previous
import jax
import jax.numpy as jnp
from jax.experimental import pallas as pl
from jax.experimental.pallas import tpu as pltpu

B = 16384
H = 64
OUT = 64
NB = 4
LN_EPS = 1e-5
BN_EPS = 1e-5
P = 2 * H
BP = B // 2
TILE = 2048


def _body(x_ref, xb_ref, lnw_ref, lnb_ref, w1_ref, b1_ref, bnw_ref, bnb_ref,
          w2_ref, b2_ref, fcw_ref, fcb_ref, o_ref, w1m_ref, wp_ref, vec_ref):
    lane = jax.lax.broadcasted_iota(jnp.int32, (P, P), 1)
    sub = jax.lax.broadcasted_iota(jnp.int32, (P, P), 0)

    @pl.when(pl.program_id(0) == 0)
    def _setup():
        bmask = (((lane ^ sub) & H) == 0).astype(jnp.float32)
        dmask = (lane == sub).astype(jnp.float32)
        m = bmask * (1.0 / H)

        def bdiag(w):
            t = jnp.concatenate([w, w], axis=1)
            return jnp.concatenate([t, t], axis=0) * bmask

        def diag(v):
            return jnp.broadcast_to(v, (P, P)) * dmask

        def tile2(v):
            return jnp.concatenate([v, v], axis=1)

        wp_ref[NB] = bdiag(fcw_ref[...])
        vec_ref[2 * NB:2 * NB + 1, :] = tile2(fcb_ref[...])

        s = 1.0 / jnp.sqrt(1.0 + BN_EPS)
        for i in range(NB):
            gp = tile2(bnw_ref[i:i + 1, :]) * s
            w1p0g = jnp.dot(bdiag(w1_ref[i]), diag(gp),
                            preferred_element_type=jnp.float32)
            w1g = jnp.dot(diag(tile2(lnw_ref[i:i + 1, :])), w1p0g,
                          preferred_element_type=jnp.float32)
            c = jnp.sum(w1g, axis=0, keepdims=True)
            w1m_ref[i] = jnp.concatenate([w1g - m * c, m], axis=1)
            wp_ref[i] = bdiag(w2_ref[i])
            d = jnp.dot(tile2(lnb_ref[i:i + 1, :]), w1p0g,
                        preferred_element_type=jnp.float32)
            d = d + tile2(b1_ref[i:i + 1, :]) * gp + tile2(bnb_ref[i:i + 1, :])
            vec_ref[i:i + 1, :] = d
            vec_ref[NB + i:NB + i + 1, :] = tile2(b2_ref[i:i + 1, :])

    x = jnp.concatenate([x_ref[0], xb_ref[0]], axis=1)
    m = w1m_ref[0][:, P:2 * P]
    for i in range(NB):
        tm = jnp.dot(x, w1m_ref[i], preferred_element_type=jnp.float32)
        q = jnp.dot(x * x, m, preferred_element_type=jnp.float32)
        t = tm[:, 0:P]
        mu = tm[:, P:2 * P]
        r = jax.lax.rsqrt(q - mu * mu + LN_EPS)
        h = r * t + vec_ref[i:i + 1, :]
        h = jnp.maximum(h, 0.0)
        h = jnp.dot(h, wp_ref[i], preferred_element_type=jnp.float32)
        x = x + h + vec_ref[NB + i:NB + i + 1, :]
    o = jnp.dot(x, wp_ref[NB], preferred_element_type=jnp.float32)
    o = o + vec_ref[2 * NB:2 * NB + 1, :]
    o_ref[0] = o[:, 0:H]
    o_ref[1] = o[:, H:P]


def kernel(x, ln_w, ln_b, w1, b1, bn_w, bn_b, w2, b2, fc_w, fc_b):
    full = lambda *shape: pl.BlockSpec(shape, lambda i: (0,) * len(shape))
    x3 = x.reshape(2, BP, H)
    out = pl.pallas_call(
        _body,
        grid=(BP // TILE,),
        in_specs=[
            pl.BlockSpec((1, TILE, H), lambda i: (0, i, 0)),
            pl.BlockSpec((1, TILE, H), lambda i: (1, i, 0)),
            full(NB, H), full(NB, H), full(NB, H, H), full(NB, H),
            full(NB, H), full(NB, H), full(NB, H, H), full(NB, H),
            full(H, OUT), full(1, OUT),
        ],
        out_specs=pl.BlockSpec((2, TILE, H), lambda i: (0, i, 0)),
        out_shape=jax.ShapeDtypeStruct((2, BP, OUT), jnp.float32),
        scratch_shapes=[
            pltpu.VMEM((NB, P, 2 * P), jnp.float32),
            pltpu.VMEM((NB + 1, P, P), jnp.float32),
            pltpu.VMEM((2 * NB + 1, P), jnp.float32),
        ],
    )(x3, x3, ln_w, ln_b, w1, b1, bn_w, bn_b, w2, b2, fc_w,
      fc_b.reshape(1, OUT))
    return out.reshape(B, OUT)
